# baseline (speedup 1.0000x reference)
import functools

import jax
import jax.numpy as jnp
from jax import lax
from jax.experimental import pallas as pl
from jax.experimental.pallas import tpu as pltpu
from jax.experimental.pallas import tpu_sc as plsc

TOP_K = 8
N_EXPERTS = 64
N_GROUP = 8
GROUP_SIZE = N_EXPERTS // N_GROUP
TOPK_GROUP = 4
SCALE = 2.5
NEG = -1e30

TBLK = 1024
MBLK = 1024
CHUNK = 128
SC_TOKENS = 8192

_IB = jax.lax.GatherScatterMode.PROMISE_IN_BOUNDS
_DNUMS = jax.lax.GatherDimensionNumbers(
    offset_dims=(), collapsed_slice_dims=(0,), start_index_map=(0,))


def _router_body(h_ref, w_ref, b_ref, idx_ref, wgt_ref):
    x = h_ref[...]
    w = w_ref[...]
    logits = jax.lax.dot_general(
        x, w, (((1,), (1,)), ((), ())),
        preferred_element_type=jnp.float32,
        precision=jax.lax.Precision.DEFAULT,
    )
    scores = jax.nn.sigmoid(logits)
    s4c = scores + b_ref[...]

    t, e = scores.shape
    lane = jax.lax.broadcasted_iota(jnp.int32, (t, e), 1)
    seg = lane % GROUP_SIZE

    def seg_roll(x_, d):
        wrap = seg >= (GROUP_SIZE - d)
        return jnp.where(wrap, jnp.roll(x_, GROUP_SIZE - d, axis=1),
                         jnp.roll(x_, -d, axis=1))

    ra = seg_roll(s4c, 4)
    a = jnp.maximum(s4c, ra)
    b = jnp.minimum(s4c, ra)
    for d in (2, 1):
        ra = seg_roll(a, d)
        rb = seg_roll(b, d)
        hi = jnp.maximum(a, ra)
        lo = jnp.minimum(a, ra)
        b = jnp.maximum(lo, jnp.maximum(b, rb))
        a = hi
    gs = a + b

    lane_f = lane.astype(jnp.float32)
    rev_f = (e - 1) - lane_f
    revgrp_f = jnp.floor(rev_f * (1.0 / GROUP_SIZE))

    avail = jnp.ones((t, e), jnp.bool_)
    sel = jnp.zeros((t, e), jnp.bool_)
    for _ in range(TOPK_GROUP):
        cur = jnp.where(avail, gs, NEG)
        mv = jnp.max(cur, axis=1, keepdims=True)
        ilr = jnp.max(jnp.where(cur == mv, rev_f, NEG), axis=1, keepdims=True)
        gsel = revgrp_f == jnp.floor(ilr * (1.0 / GROUP_SIZE))
        sel = jnp.logical_or(sel, gsel)
        avail = jnp.logical_and(avail, jnp.logical_not(gsel))

    ms = jnp.where(sel, s4c, 0.0)
    idx_cols = []
    wgt_cols = []
    for _ in range(TOP_K):
        mv = jnp.max(ms, axis=1, keepdims=True)
        ilr = jnp.max(jnp.where(ms == mv, rev_f, NEG), axis=1, keepdims=True)
        ms = jnp.where(rev_f == ilr, NEG, ms)
        idx_cols.append(ilr)
        wgt_cols.append(mv)

    wsum = functools.reduce(jnp.add, wgt_cols)
    inv = 1.0 / (wsum + 1e-20)

    lane_k = jax.lax.broadcasted_iota(jnp.int32, (t, TOP_K), 1)
    idx_out = jnp.zeros((t, TOP_K), jnp.float32)
    wgt_out = jnp.zeros((t, TOP_K), jnp.float32)
    for k in range(TOP_K):
        sel_k = lane_k == k
        idx_out = jnp.where(sel_k, idx_cols[k], idx_out)
        wgt_out = jnp.where(sel_k, wgt_cols[k], wgt_out)
    wgt_out = (wgt_out * inv) * SCALE

    idx_ref[...] = ((e - 1) - idx_out).astype(jnp.int32)
    wgt_ref[...] = wgt_out


def _fused_tc(hf, weight, bias2):
    t, h = hf.shape
    return pl.pallas_call(
        _router_body,
        grid=(t // TBLK,),
        in_specs=[
            pl.BlockSpec((TBLK, h), lambda i: (i, 0)),
            pl.BlockSpec((N_EXPERTS, h), lambda i: (0, 0)),
            pl.BlockSpec((1, N_EXPERTS), lambda i: (0, 0)),
        ],
        out_specs=[
            pl.BlockSpec((TBLK, TOP_K), lambda i: (i, 0)),
            pl.BlockSpec((TBLK, TOP_K), lambda i: (i, 0)),
        ],
        out_shape=[
            jax.ShapeDtypeStruct((t, TOP_K), jnp.int32),
            jax.ShapeDtypeStruct((t, TOP_K), jnp.float32),
        ],
        compiler_params=pltpu.CompilerParams(
            dimension_semantics=("arbitrary",),
        ),
    )(hf, weight, bias2)


def _matmul_body(h_ref, w_ref, s_ref):
    logits = jax.lax.dot_general(
        h_ref[...], w_ref[...], (((1,), (1,)), ((), ())),
        preferred_element_type=jnp.float32,
        precision=jax.lax.Precision.DEFAULT,
    )
    s_ref[...] = jax.nn.sigmoid(logits)


def _scores_tc(hf, weight):
    t, h = hf.shape
    return pl.pallas_call(
        _matmul_body,
        grid=(t // MBLK,),
        in_specs=[
            pl.BlockSpec((MBLK, h), lambda i: (i, 0)),
            pl.BlockSpec((N_EXPERTS, h), lambda i: (0, 0)),
        ],
        out_specs=pl.BlockSpec((MBLK, N_EXPERTS), lambda i: (i, 0)),
        out_shape=jax.ShapeDtypeStruct((t, N_EXPERTS), jnp.float32),
        compiler_params=pltpu.CompilerParams(
            dimension_semantics=("arbitrary",),
        ),
    )(hf, weight)


def _take(v, p):
    return jax.lax.gather(v, p[:, None], dimension_numbers=_DNUMS,
                          slice_sizes=(1,), mode=_IB)


def _sc_router(t):
    n_workers = 32
    tpw = t // n_workers
    n_chunks = tpw // CHUNK
    mesh = plsc.VectorSubcoreMesh(core_axis_name="c", subcore_axis_name="s")

    @functools.partial(
        pl.kernel,
        out_type=[
            jax.ShapeDtypeStruct((t * TOP_K,), jnp.int32),
            jax.ShapeDtypeStruct((t * TOP_K,), jnp.float32),
        ],
        mesh=mesh,
        scratch_types=[
            pltpu.VMEM((CHUNK, N_EXPERTS), jnp.float32),
            pltpu.VMEM((64,), jnp.float32),
            pltpu.VMEM((CHUNK * TOP_K + 16,), jnp.int32),
            pltpu.VMEM((CHUNK * TOP_K + 16,), jnp.float32),
        ],
    )
    def route(scores_hbm, bias_hbm, idx_hbm, wgt_hbm, s_v, b_v, i_v, w_v):
        wid = lax.axis_index("s") * 2 + lax.axis_index("c")
        base = wid * tpw

        pltpu.sync_copy(bias_hbm, b_v)
        bias_vregs = [b_v[pl.ds(16 * k, 16)] for k in range(4)]

        iota = lax.iota(jnp.int32, 16)
        iota_f = iota.astype(jnp.float32)
        perms = {d: jnp.bitwise_xor(iota, d) for d in (1, 2, 4, 8)}
        pat08 = (iota & 1) << 3
        half = iota >> 3
        g_of_lane = iota

        def group_top2(v):
            p = _take(v, perms[4])
            a = jnp.maximum(v, p)
            b = jnp.minimum(v, p)
            for d in (2, 1):
                pa = _take(a, perms[d])
                pb = _take(b, perms[d])
                hi = jnp.maximum(a, pa)
                lo = jnp.minimum(a, pa)
                b = jnp.maximum(lo, jnp.maximum(b, pb))
                a = hi
            return a + b

        def token_body(tok, _):
            v = [s_v[tok, pl.ds(16 * k, 16)] for k in range(4)]
            s4c = [v[k] + bias_vregs[k] for k in range(4)]

            gs = [group_top2(s4c[k]) for k in range(4)]
            g8 = _take(gs[0], pat08)
            for k in range(1, 4):
                pk = _take(gs[k], pat08)
                g8 = jnp.where((iota >> 1) == k, pk, g8)

            cnt = jnp.zeros((16,), jnp.float32)
            for k in range(1, N_GROUP):
                pat = (iota + k) & (N_GROUP - 1)
                sg = _take(g8, pat)
                lower = pat < g_of_lane
                beat = (sg > g8) | ((sg == g8) & lower)
                cnt = cnt + jnp.where(beat, 1.0, 0.0)
            selg = jnp.where(cnt < float(TOPK_GROUP), 1.0, 0.0)

            ms = []
            for k in range(4):
                mk = _take(selg, 2 * k + half)
                ms.append(jnp.where(mk > 0.5, s4c[k], 0.0))

            rev = [63.0 - (16.0 * k + iota_f) for k in range(4)]
            res_i = jnp.zeros((16,), jnp.float32)
            res_w = jnp.zeros((16,), jnp.float32)
            for r in range(TOP_K):
                m = jnp.maximum(jnp.maximum(ms[0], ms[1]),
                                jnp.maximum(ms[2], ms[3]))
                for d in (8, 4, 2, 1):
                    m = jnp.maximum(m, _take(m, perms[d]))
                c = jnp.full((16,), NEG, jnp.float32)
                for k in range(4):
                    c = jnp.maximum(c, jnp.where(ms[k] == m, rev[k], NEG))
                for d in (8, 4, 2, 1):
                    c = jnp.maximum(c, _take(c, perms[d]))
                ms = [jnp.where(rev[k] == c, NEG, ms[k]) for k in range(4)]
                res_i = jnp.where(iota == r, 63.0 - c, res_i)
                res_w = jnp.where(iota == r, m, res_w)

            tot = jnp.where(iota < TOP_K, res_w, 0.0)
            for d in (8, 4, 2, 1):
                tot = tot + _take(tot, perms[d])
            res_w = (res_w / (tot + 1e-20)) * SCALE

            i_v[pl.ds(tok * TOP_K, 16)] = res_i.astype(jnp.int32)
            w_v[pl.ds(tok * TOP_K, 16)] = res_w
            return 0

        for ci in range(n_chunks):
            off = base + ci * CHUNK
            pltpu.sync_copy(scores_hbm.at[pl.ds(off, CHUNK)], s_v)
            lax.fori_loop(0, CHUNK, token_body, 0)
            pltpu.sync_copy(i_v.at[pl.ds(0, CHUNK * TOP_K)],
                            idx_hbm.at[pl.ds(off * TOP_K, CHUNK * TOP_K)])
            pltpu.sync_copy(w_v.at[pl.ds(0, CHUNK * TOP_K)],
                            wgt_hbm.at[pl.ds(off * TOP_K, CHUNK * TOP_K)])

    return route


def kernel(hidden_states, weight, e_score_correction_bias):
    b, s, h = hidden_states.shape
    hf = hidden_states.reshape(-1, h).astype(jnp.float32)
    t = hf.shape[0]
    w32 = weight.astype(jnp.float32)
    bias = e_score_correction_bias.astype(jnp.float32)
    bias2 = bias.reshape(1, N_EXPERTS)

    t_sc = SC_TOKENS
    scores_sc = _scores_tc(hf[:t_sc], w32)
    idx_sc, wgt_sc = _sc_router(t_sc)(scores_sc, bias)
    idx_tc, wgt_tc = _fused_tc(hf[t_sc:], w32, bias2)

    idx = jnp.concatenate([idx_sc.reshape(t_sc, TOP_K), idx_tc], axis=0)
    wgt = jnp.concatenate([wgt_sc.reshape(t_sc, TOP_K), wgt_tc], axis=0)
    return idx, wgt


# speedup vs baseline: 2.0896x; 2.0896x over previous
import functools

import jax
import jax.numpy as jnp
from jax import lax
from jax.experimental import pallas as pl
from jax.experimental.pallas import tpu as pltpu
from jax.experimental.pallas import tpu_sc as plsc

TOP_K = 8
N_EXPERTS = 64
N_GROUP = 8
GROUP_SIZE = N_EXPERTS // N_GROUP
TOPK_GROUP = 4
SCALE = 2.5
NEG = -1e30

TBLK = 1024
MBLK = 1024
CHUNK = 128
SC_TOKENS = 8192

_IB = jax.lax.GatherScatterMode.PROMISE_IN_BOUNDS
_DNUMS = jax.lax.GatherDimensionNumbers(
    offset_dims=(), collapsed_slice_dims=(0,), start_index_map=(0,))


def _router_body(h_ref, w_ref, b_ref, idx_ref, wgt_ref):
    x = h_ref[...]
    w = w_ref[...]
    logits = jax.lax.dot_general(
        x, w, (((1,), (1,)), ((), ())),
        preferred_element_type=jnp.float32,
        precision=jax.lax.Precision.DEFAULT,
    )
    scores = jax.nn.sigmoid(logits)
    s4c = scores + b_ref[...]

    t, e = scores.shape
    lane = jax.lax.broadcasted_iota(jnp.int32, (t, e), 1)
    seg = lane % GROUP_SIZE

    def seg_roll(x_, d):
        wrap = seg >= (GROUP_SIZE - d)
        return jnp.where(wrap, jnp.roll(x_, GROUP_SIZE - d, axis=1),
                         jnp.roll(x_, -d, axis=1))

    ra = seg_roll(s4c, 4)
    a = jnp.maximum(s4c, ra)
    b = jnp.minimum(s4c, ra)
    for d in (2, 1):
        ra = seg_roll(a, d)
        rb = seg_roll(b, d)
        hi = jnp.maximum(a, ra)
        lo = jnp.minimum(a, ra)
        b = jnp.maximum(lo, jnp.maximum(b, rb))
        a = hi
    gs = a + b

    lane_f = lane.astype(jnp.float32)
    rev_f = (e - 1) - lane_f
    revgrp_f = jnp.floor(rev_f * (1.0 / GROUP_SIZE))

    avail = jnp.ones((t, e), jnp.bool_)
    sel = jnp.zeros((t, e), jnp.bool_)
    for _ in range(TOPK_GROUP):
        cur = jnp.where(avail, gs, NEG)
        mv = jnp.max(cur, axis=1, keepdims=True)
        ilr = jnp.max(jnp.where(cur == mv, rev_f, NEG), axis=1, keepdims=True)
        gsel = revgrp_f == jnp.floor(ilr * (1.0 / GROUP_SIZE))
        sel = jnp.logical_or(sel, gsel)
        avail = jnp.logical_and(avail, jnp.logical_not(gsel))

    ms = jnp.where(sel, s4c, 0.0)
    idx_cols = []
    wgt_cols = []
    for _ in range(TOP_K):
        mv = jnp.max(ms, axis=1, keepdims=True)
        ilr = jnp.max(jnp.where(ms == mv, rev_f, NEG), axis=1, keepdims=True)
        ms = jnp.where(rev_f == ilr, NEG, ms)
        idx_cols.append(ilr)
        wgt_cols.append(mv)

    wsum = functools.reduce(jnp.add, wgt_cols)
    inv = 1.0 / (wsum + 1e-20)

    lane_k = jax.lax.broadcasted_iota(jnp.int32, (t, TOP_K), 1)
    idx_out = jnp.zeros((t, TOP_K), jnp.float32)
    wgt_out = jnp.zeros((t, TOP_K), jnp.float32)
    for k in range(TOP_K):
        sel_k = lane_k == k
        idx_out = jnp.where(sel_k, idx_cols[k], idx_out)
        wgt_out = jnp.where(sel_k, wgt_cols[k], wgt_out)
    wgt_out = (wgt_out * inv) * SCALE

    idx_ref[...] = ((e - 1) - idx_out).astype(jnp.int32)
    wgt_ref[...] = wgt_out


def _fused_tc(hf, weight, bias2, skip_blocks):
    t, h = hf.shape
    off = skip_blocks
    return pl.pallas_call(
        _router_body,
        grid=(t // TBLK - skip_blocks,),
        in_specs=[
            pl.BlockSpec((TBLK, h), lambda i: (i + off, 0)),
            pl.BlockSpec((N_EXPERTS, h), lambda i: (0, 0)),
            pl.BlockSpec((1, N_EXPERTS), lambda i: (0, 0)),
        ],
        out_specs=[
            pl.BlockSpec((TBLK, TOP_K), lambda i: (i + off, 0)),
            pl.BlockSpec((TBLK, TOP_K), lambda i: (i + off, 0)),
        ],
        out_shape=[
            jax.ShapeDtypeStruct((t, TOP_K), jnp.int32),
            jax.ShapeDtypeStruct((t, TOP_K), jnp.float32),
        ],
        compiler_params=pltpu.CompilerParams(
            dimension_semantics=("arbitrary",),
        ),
    )(hf, weight, bias2)


def _matmul_body(h_ref, w_ref, s_ref):
    logits = jax.lax.dot_general(
        h_ref[...], w_ref[...], (((1,), (1,)), ((), ())),
        preferred_element_type=jnp.float32,
        precision=jax.lax.Precision.DEFAULT,
    )
    s_ref[...] = jax.nn.sigmoid(logits)


def _scores_tc(hf, weight, n_blocks):
    t, h = hf.shape
    return pl.pallas_call(
        _matmul_body,
        grid=(n_blocks,),
        in_specs=[
            pl.BlockSpec((MBLK, h), lambda i: (i, 0)),
            pl.BlockSpec((N_EXPERTS, h), lambda i: (0, 0)),
        ],
        out_specs=pl.BlockSpec((MBLK, N_EXPERTS), lambda i: (i, 0)),
        out_shape=jax.ShapeDtypeStruct((n_blocks * MBLK, N_EXPERTS),
                                       jnp.float32),
        compiler_params=pltpu.CompilerParams(
            dimension_semantics=("arbitrary",),
        ),
    )(hf, weight)


def _take(v, p):
    return jax.lax.gather(v, p[:, None], dimension_numbers=_DNUMS,
                          slice_sizes=(1,), mode=_IB)


def _sc_router(t):
    n_workers = 32
    tpw = t // n_workers
    n_chunks = tpw // CHUNK
    mesh = plsc.VectorSubcoreMesh(core_axis_name="c", subcore_axis_name="s")

    @functools.partial(
        pl.kernel,
        out_type=[
            jax.ShapeDtypeStruct((t * TOP_K,), jnp.int32),
            jax.ShapeDtypeStruct((t * TOP_K,), jnp.float32),
        ],
        mesh=mesh,
        scratch_types=[
            pltpu.VMEM((CHUNK, N_EXPERTS), jnp.float32),
            pltpu.VMEM((64,), jnp.float32),
            pltpu.VMEM((CHUNK * TOP_K + 16,), jnp.int32),
            pltpu.VMEM((CHUNK * TOP_K + 16,), jnp.float32),
        ],
    )
    def route(scores_hbm, bias_hbm, idx_hbm, wgt_hbm, s_v, b_v, i_v, w_v):
        wid = lax.axis_index("s") * 2 + lax.axis_index("c")
        base = wid * tpw

        pltpu.sync_copy(bias_hbm, b_v)
        bias_vregs = [b_v[pl.ds(16 * k, 16)] for k in range(4)]

        iota = lax.iota(jnp.int32, 16)
        iota_f = iota.astype(jnp.float32)
        perms = {d: jnp.bitwise_xor(iota, d) for d in (1, 2, 4, 8)}
        pat08 = (iota & 1) << 3
        half = iota >> 3
        g_of_lane = iota

        def group_top2(v):
            p = _take(v, perms[4])
            a = jnp.maximum(v, p)
            b = jnp.minimum(v, p)
            for d in (2, 1):
                pa = _take(a, perms[d])
                pb = _take(b, perms[d])
                hi = jnp.maximum(a, pa)
                lo = jnp.minimum(a, pa)
                b = jnp.maximum(lo, jnp.maximum(b, pb))
                a = hi
            return a + b

        def token_body(tok, _):
            v = [s_v[tok, pl.ds(16 * k, 16)] for k in range(4)]
            s4c = [v[k] + bias_vregs[k] for k in range(4)]

            gs = [group_top2(s4c[k]) for k in range(4)]
            g8 = _take(gs[0], pat08)
            for k in range(1, 4):
                pk = _take(gs[k], pat08)
                g8 = jnp.where((iota >> 1) == k, pk, g8)

            cnt = jnp.zeros((16,), jnp.float32)
            for k in range(1, N_GROUP):
                pat = (iota + k) & (N_GROUP - 1)
                sg = _take(g8, pat)
                lower = pat < g_of_lane
                beat = (sg > g8) | ((sg == g8) & lower)
                cnt = cnt + jnp.where(beat, 1.0, 0.0)
            selg = jnp.where(cnt < float(TOPK_GROUP), 1.0, 0.0)

            ms = []
            for k in range(4):
                mk = _take(selg, 2 * k + half)
                ms.append(jnp.where(mk > 0.5, s4c[k], 0.0))

            rev = [63.0 - (16.0 * k + iota_f) for k in range(4)]
            res_i = jnp.zeros((16,), jnp.float32)
            res_w = jnp.zeros((16,), jnp.float32)
            for r in range(TOP_K):
                m = jnp.maximum(jnp.maximum(ms[0], ms[1]),
                                jnp.maximum(ms[2], ms[3]))
                for d in (8, 4, 2, 1):
                    m = jnp.maximum(m, _take(m, perms[d]))
                c = jnp.full((16,), NEG, jnp.float32)
                for k in range(4):
                    c = jnp.maximum(c, jnp.where(ms[k] == m, rev[k], NEG))
                for d in (8, 4, 2, 1):
                    c = jnp.maximum(c, _take(c, perms[d]))
                ms = [jnp.where(rev[k] == c, NEG, ms[k]) for k in range(4)]
                res_i = jnp.where(iota == r, 63.0 - c, res_i)
                res_w = jnp.where(iota == r, m, res_w)

            tot = jnp.where(iota < TOP_K, res_w, 0.0)
            for d in (8, 4, 2, 1):
                tot = tot + _take(tot, perms[d])
            res_w = (res_w / (tot + 1e-20)) * SCALE

            i_v[pl.ds(tok * TOP_K, 16)] = res_i.astype(jnp.int32)
            w_v[pl.ds(tok * TOP_K, 16)] = res_w
            return 0

        for ci in range(n_chunks):
            off = base + ci * CHUNK
            pltpu.sync_copy(scores_hbm.at[pl.ds(off, CHUNK)], s_v)
            lax.fori_loop(0, CHUNK, token_body, 0)
            pltpu.sync_copy(i_v.at[pl.ds(0, CHUNK * TOP_K)],
                            idx_hbm.at[pl.ds(off * TOP_K, CHUNK * TOP_K)])
            pltpu.sync_copy(w_v.at[pl.ds(0, CHUNK * TOP_K)],
                            wgt_hbm.at[pl.ds(off * TOP_K, CHUNK * TOP_K)])

    return route


def kernel(hidden_states, weight, e_score_correction_bias):
    b, s, h = hidden_states.shape
    hf = hidden_states.reshape(-1, h).astype(jnp.float32)
    t = hf.shape[0]
    w32 = weight.astype(jnp.float32)
    bias = e_score_correction_bias.astype(jnp.float32)
    bias2 = bias.reshape(1, N_EXPERTS)

    t_sc = SC_TOKENS
    scores_sc = _scores_tc(hf, w32, t_sc // MBLK)
    idx_sc, wgt_sc = _sc_router(t_sc)(scores_sc, bias)
    idx_tc, wgt_tc = _fused_tc(hf, w32, bias2, t_sc // TBLK)

    idx = jax.lax.dynamic_update_slice(
        idx_tc, idx_sc.reshape(t_sc, TOP_K), (0, 0))
    wgt = jax.lax.dynamic_update_slice(
        wgt_tc, wgt_sc.reshape(t_sc, TOP_K), (0, 0))
    return idx, wgt


# speedup vs baseline: 2.1160x; 1.0126x over previous
import functools

import jax
import jax.numpy as jnp
from jax import lax
from jax.experimental import pallas as pl
from jax.experimental.pallas import tpu as pltpu
from jax.experimental.pallas import tpu_sc as plsc

TOP_K = 8
N_EXPERTS = 64
N_GROUP = 8
GROUP_SIZE = N_EXPERTS // N_GROUP
TOPK_GROUP = 4
SCALE = 2.5
NEG = -1e30

TBLK = 1024
MBLK = 1024
CHUNK = 128
SC_TOKENS = 12288

_IB = jax.lax.GatherScatterMode.PROMISE_IN_BOUNDS
_DNUMS = jax.lax.GatherDimensionNumbers(
    offset_dims=(), collapsed_slice_dims=(0,), start_index_map=(0,))


def _router_body(h_ref, w_ref, b_ref, idx_ref, wgt_ref):
    x = h_ref[...]
    w = w_ref[...]
    logits = jax.lax.dot_general(
        x, w, (((1,), (1,)), ((), ())),
        preferred_element_type=jnp.float32,
        precision=jax.lax.Precision.DEFAULT,
    )
    scores = jax.nn.sigmoid(logits)
    s4c = scores + b_ref[...]

    t, e = scores.shape
    lane = jax.lax.broadcasted_iota(jnp.int32, (t, e), 1)
    seg = lane % GROUP_SIZE

    def seg_roll(x_, d):
        wrap = seg >= (GROUP_SIZE - d)
        return jnp.where(wrap, jnp.roll(x_, GROUP_SIZE - d, axis=1),
                         jnp.roll(x_, -d, axis=1))

    ra = seg_roll(s4c, 4)
    a = jnp.maximum(s4c, ra)
    b = jnp.minimum(s4c, ra)
    for d in (2, 1):
        ra = seg_roll(a, d)
        rb = seg_roll(b, d)
        hi = jnp.maximum(a, ra)
        lo = jnp.minimum(a, ra)
        b = jnp.maximum(lo, jnp.maximum(b, rb))
        a = hi
    gs = a + b

    lane_f = lane.astype(jnp.float32)
    rev_f = (e - 1) - lane_f
    revgrp_f = jnp.floor(rev_f * (1.0 / GROUP_SIZE))

    avail = jnp.ones((t, e), jnp.bool_)
    sel = jnp.zeros((t, e), jnp.bool_)
    for _ in range(TOPK_GROUP):
        cur = jnp.where(avail, gs, NEG)
        mv = jnp.max(cur, axis=1, keepdims=True)
        ilr = jnp.max(jnp.where(cur == mv, rev_f, NEG), axis=1, keepdims=True)
        gsel = revgrp_f == jnp.floor(ilr * (1.0 / GROUP_SIZE))
        sel = jnp.logical_or(sel, gsel)
        avail = jnp.logical_and(avail, jnp.logical_not(gsel))

    ms = jnp.where(sel, s4c, 0.0)
    idx_cols = []
    wgt_cols = []
    for _ in range(TOP_K):
        mv = jnp.max(ms, axis=1, keepdims=True)
        ilr = jnp.max(jnp.where(ms == mv, rev_f, NEG), axis=1, keepdims=True)
        ms = jnp.where(rev_f == ilr, NEG, ms)
        idx_cols.append(ilr)
        wgt_cols.append(mv)

    wsum = functools.reduce(jnp.add, wgt_cols)
    inv = 1.0 / (wsum + 1e-20)

    lane_k = jax.lax.broadcasted_iota(jnp.int32, (t, TOP_K), 1)
    idx_out = jnp.zeros((t, TOP_K), jnp.float32)
    wgt_out = jnp.zeros((t, TOP_K), jnp.float32)
    for k in range(TOP_K):
        sel_k = lane_k == k
        idx_out = jnp.where(sel_k, idx_cols[k], idx_out)
        wgt_out = jnp.where(sel_k, wgt_cols[k], wgt_out)
    wgt_out = (wgt_out * inv) * SCALE

    idx_ref[...] = ((e - 1) - idx_out).astype(jnp.int32)
    wgt_ref[...] = wgt_out


def _fused_tc(hf, weight, bias2, skip_blocks):
    t, h = hf.shape
    off = skip_blocks
    return pl.pallas_call(
        _router_body,
        grid=(t // TBLK - skip_blocks,),
        in_specs=[
            pl.BlockSpec((TBLK, h), lambda i: (i + off, 0)),
            pl.BlockSpec((N_EXPERTS, h), lambda i: (0, 0)),
            pl.BlockSpec((1, N_EXPERTS), lambda i: (0, 0)),
        ],
        out_specs=[
            pl.BlockSpec((TBLK, TOP_K), lambda i: (i + off, 0)),
            pl.BlockSpec((TBLK, TOP_K), lambda i: (i + off, 0)),
        ],
        out_shape=[
            jax.ShapeDtypeStruct((t, TOP_K), jnp.int32),
            jax.ShapeDtypeStruct((t, TOP_K), jnp.float32),
        ],
        compiler_params=pltpu.CompilerParams(
            dimension_semantics=("arbitrary",),
        ),
    )(hf, weight, bias2)


def _matmul_body(h_ref, w_ref, s_ref):
    logits = jax.lax.dot_general(
        h_ref[...], w_ref[...], (((1,), (1,)), ((), ())),
        preferred_element_type=jnp.float32,
        precision=jax.lax.Precision.DEFAULT,
    )
    s_ref[...] = jax.nn.sigmoid(logits)


def _scores_tc(hf, weight, n_blocks):
    t, h = hf.shape
    return pl.pallas_call(
        _matmul_body,
        grid=(n_blocks,),
        in_specs=[
            pl.BlockSpec((MBLK, h), lambda i: (i, 0)),
            pl.BlockSpec((N_EXPERTS, h), lambda i: (0, 0)),
        ],
        out_specs=pl.BlockSpec((MBLK, N_EXPERTS), lambda i: (i, 0)),
        out_shape=jax.ShapeDtypeStruct((n_blocks * MBLK, N_EXPERTS),
                                       jnp.float32),
        compiler_params=pltpu.CompilerParams(
            dimension_semantics=("arbitrary",),
        ),
    )(hf, weight)


def _take(v, p):
    return jax.lax.gather(v, p[:, None], dimension_numbers=_DNUMS,
                          slice_sizes=(1,), mode=_IB)


def _sc_router(t):
    n_workers = 32
    tpw = t // n_workers
    n_chunks = tpw // CHUNK
    mesh = plsc.VectorSubcoreMesh(core_axis_name="c", subcore_axis_name="s")

    @functools.partial(
        pl.kernel,
        out_type=[
            jax.ShapeDtypeStruct((t * TOP_K,), jnp.int32),
            jax.ShapeDtypeStruct((t * TOP_K,), jnp.float32),
        ],
        mesh=mesh,
        scratch_types=[
            pltpu.VMEM((CHUNK, N_EXPERTS), jnp.float32),
            pltpu.VMEM((64,), jnp.float32),
            pltpu.VMEM((CHUNK * TOP_K + 16,), jnp.int32),
            pltpu.VMEM((CHUNK * TOP_K + 16,), jnp.float32),
        ],
    )
    def route(scores_hbm, bias_hbm, idx_hbm, wgt_hbm, s_v, b_v, i_v, w_v):
        wid = lax.axis_index("s") * 2 + lax.axis_index("c")
        base = wid * tpw

        pltpu.sync_copy(bias_hbm, b_v)
        bias_vregs = [b_v[pl.ds(16 * k, 16)] for k in range(4)]

        iota = lax.iota(jnp.int32, 16)
        iota_f = iota.astype(jnp.float32)
        perms = {d: jnp.bitwise_xor(iota, d) for d in (1, 2, 4, 8)}
        pat08 = (iota & 1) << 3
        half = iota >> 3
        g_of_lane = iota

        def group_top2(v):
            p = _take(v, perms[4])
            a = jnp.maximum(v, p)
            b = jnp.minimum(v, p)
            for d in (2, 1):
                pa = _take(a, perms[d])
                pb = _take(b, perms[d])
                hi = jnp.maximum(a, pa)
                lo = jnp.minimum(a, pa)
                b = jnp.maximum(lo, jnp.maximum(b, pb))
                a = hi
            return a + b

        def token_body(tok, _):
            v = [s_v[tok, pl.ds(16 * k, 16)] for k in range(4)]
            s4c = [v[k] + bias_vregs[k] for k in range(4)]

            gs = [group_top2(s4c[k]) for k in range(4)]
            g8 = _take(gs[0], pat08)
            for k in range(1, 4):
                pk = _take(gs[k], pat08)
                g8 = jnp.where((iota >> 1) == k, pk, g8)

            cnt = jnp.zeros((16,), jnp.float32)
            for k in range(1, N_GROUP):
                pat = (iota + k) & (N_GROUP - 1)
                sg = _take(g8, pat)
                lower = pat < g_of_lane
                beat = (sg > g8) | ((sg == g8) & lower)
                cnt = cnt + jnp.where(beat, 1.0, 0.0)
            selg = jnp.where(cnt < float(TOPK_GROUP), 1.0, 0.0)

            ms = []
            for k in range(4):
                mk = _take(selg, 2 * k + half)
                ms.append(jnp.where(mk > 0.5, s4c[k], 0.0))

            rev = [63.0 - (16.0 * k + iota_f) for k in range(4)]
            res_i = jnp.zeros((16,), jnp.float32)
            res_w = jnp.zeros((16,), jnp.float32)
            for r in range(TOP_K):
                m = jnp.maximum(jnp.maximum(ms[0], ms[1]),
                                jnp.maximum(ms[2], ms[3]))
                for d in (8, 4, 2, 1):
                    m = jnp.maximum(m, _take(m, perms[d]))
                c = jnp.full((16,), NEG, jnp.float32)
                for k in range(4):
                    c = jnp.maximum(c, jnp.where(ms[k] == m, rev[k], NEG))
                for d in (8, 4, 2, 1):
                    c = jnp.maximum(c, _take(c, perms[d]))
                ms = [jnp.where(rev[k] == c, NEG, ms[k]) for k in range(4)]
                res_i = jnp.where(iota == r, 63.0 - c, res_i)
                res_w = jnp.where(iota == r, m, res_w)

            tot = jnp.where(iota < TOP_K, res_w, 0.0)
            for d in (8, 4, 2, 1):
                tot = tot + _take(tot, perms[d])
            res_w = (res_w / (tot + 1e-20)) * SCALE

            i_v[pl.ds(tok * TOP_K, 16)] = res_i.astype(jnp.int32)
            w_v[pl.ds(tok * TOP_K, 16)] = res_w
            return 0

        for ci in range(n_chunks):
            off = base + ci * CHUNK
            pltpu.sync_copy(scores_hbm.at[pl.ds(off, CHUNK)], s_v)
            lax.fori_loop(0, CHUNK, token_body, 0)
            pltpu.sync_copy(i_v.at[pl.ds(0, CHUNK * TOP_K)],
                            idx_hbm.at[pl.ds(off * TOP_K, CHUNK * TOP_K)])
            pltpu.sync_copy(w_v.at[pl.ds(0, CHUNK * TOP_K)],
                            wgt_hbm.at[pl.ds(off * TOP_K, CHUNK * TOP_K)])

    return route


def kernel(hidden_states, weight, e_score_correction_bias):
    b, s, h = hidden_states.shape
    hf = hidden_states.reshape(-1, h).astype(jnp.float32)
    t = hf.shape[0]
    w32 = weight.astype(jnp.float32)
    bias = e_score_correction_bias.astype(jnp.float32)
    bias2 = bias.reshape(1, N_EXPERTS)

    t_sc = SC_TOKENS
    scores_sc = _scores_tc(hf, w32, t_sc // MBLK)
    idx_sc, wgt_sc = _sc_router(t_sc)(scores_sc, bias)
    idx_tc, wgt_tc = _fused_tc(hf, w32, bias2, t_sc // TBLK)

    idx = jax.lax.dynamic_update_slice(
        idx_tc, idx_sc.reshape(t_sc, TOP_K), (0, 0))
    wgt = jax.lax.dynamic_update_slice(
        wgt_tc, wgt_sc.reshape(t_sc, TOP_K), (0, 0))
    return idx, wgt


# speedup vs baseline: 2.1858x; 1.0330x over previous
import functools

import jax
import jax.numpy as jnp
from jax import lax
from jax.experimental import pallas as pl
from jax.experimental.pallas import tpu as pltpu
from jax.experimental.pallas import tpu_sc as plsc

TOP_K = 8
N_EXPERTS = 64
N_GROUP = 8
GROUP_SIZE = N_EXPERTS // N_GROUP
TOPK_GROUP = 4
SCALE = 2.5
NEG = -1e30

TBLK = 1024
MBLK = 1024
CHUNK = 128
SC_TOKENS = 16384

_IB = jax.lax.GatherScatterMode.PROMISE_IN_BOUNDS
_DNUMS = jax.lax.GatherDimensionNumbers(
    offset_dims=(), collapsed_slice_dims=(0,), start_index_map=(0,))


def _router_body(h_ref, w_ref, b_ref, idx_ref, wgt_ref):
    x = h_ref[...]
    w = w_ref[...]
    logits = jax.lax.dot_general(
        x, w, (((1,), (1,)), ((), ())),
        preferred_element_type=jnp.float32,
        precision=jax.lax.Precision.DEFAULT,
    )
    scores = jax.nn.sigmoid(logits)
    s4c = scores + b_ref[...]

    t, e = scores.shape
    lane = jax.lax.broadcasted_iota(jnp.int32, (t, e), 1)
    seg = lane % GROUP_SIZE

    def seg_roll(x_, d):
        wrap = seg >= (GROUP_SIZE - d)
        return jnp.where(wrap, jnp.roll(x_, GROUP_SIZE - d, axis=1),
                         jnp.roll(x_, -d, axis=1))

    ra = seg_roll(s4c, 4)
    a = jnp.maximum(s4c, ra)
    b = jnp.minimum(s4c, ra)
    for d in (2, 1):
        ra = seg_roll(a, d)
        rb = seg_roll(b, d)
        hi = jnp.maximum(a, ra)
        lo = jnp.minimum(a, ra)
        b = jnp.maximum(lo, jnp.maximum(b, rb))
        a = hi
    gs = a + b

    lane_f = lane.astype(jnp.float32)
    rev_f = (e - 1) - lane_f
    revgrp_f = jnp.floor(rev_f * (1.0 / GROUP_SIZE))

    avail = jnp.ones((t, e), jnp.bool_)
    sel = jnp.zeros((t, e), jnp.bool_)
    for _ in range(TOPK_GROUP):
        cur = jnp.where(avail, gs, NEG)
        mv = jnp.max(cur, axis=1, keepdims=True)
        ilr = jnp.max(jnp.where(cur == mv, rev_f, NEG), axis=1, keepdims=True)
        gsel = revgrp_f == jnp.floor(ilr * (1.0 / GROUP_SIZE))
        sel = jnp.logical_or(sel, gsel)
        avail = jnp.logical_and(avail, jnp.logical_not(gsel))

    ms = jnp.where(sel, s4c, 0.0)
    idx_cols = []
    wgt_cols = []
    for _ in range(TOP_K):
        mv = jnp.max(ms, axis=1, keepdims=True)
        ilr = jnp.max(jnp.where(ms == mv, rev_f, NEG), axis=1, keepdims=True)
        ms = jnp.where(rev_f == ilr, NEG, ms)
        idx_cols.append(ilr)
        wgt_cols.append(mv)

    wsum = functools.reduce(jnp.add, wgt_cols)
    inv = 1.0 / (wsum + 1e-20)

    lane_k = jax.lax.broadcasted_iota(jnp.int32, (t, TOP_K), 1)
    idx_out = jnp.zeros((t, TOP_K), jnp.float32)
    wgt_out = jnp.zeros((t, TOP_K), jnp.float32)
    for k in range(TOP_K):
        sel_k = lane_k == k
        idx_out = jnp.where(sel_k, idx_cols[k], idx_out)
        wgt_out = jnp.where(sel_k, wgt_cols[k], wgt_out)
    wgt_out = (wgt_out * inv) * SCALE

    idx_ref[...] = ((e - 1) - idx_out).astype(jnp.int32)
    wgt_ref[...] = wgt_out


def _fused_tc(hf, weight, bias2, skip_blocks):
    t, h = hf.shape
    off = skip_blocks
    return pl.pallas_call(
        _router_body,
        grid=(t // TBLK - skip_blocks,),
        in_specs=[
            pl.BlockSpec((TBLK, h), lambda i: (i + off, 0)),
            pl.BlockSpec((N_EXPERTS, h), lambda i: (0, 0)),
            pl.BlockSpec((1, N_EXPERTS), lambda i: (0, 0)),
        ],
        out_specs=[
            pl.BlockSpec((TBLK, TOP_K), lambda i: (i + off, 0)),
            pl.BlockSpec((TBLK, TOP_K), lambda i: (i + off, 0)),
        ],
        out_shape=[
            jax.ShapeDtypeStruct((t, TOP_K), jnp.int32),
            jax.ShapeDtypeStruct((t, TOP_K), jnp.float32),
        ],
        compiler_params=pltpu.CompilerParams(
            dimension_semantics=("arbitrary",),
        ),
    )(hf, weight, bias2)


def _matmul_body(h_ref, w_ref, s_ref):
    logits = jax.lax.dot_general(
        h_ref[...], w_ref[...], (((1,), (1,)), ((), ())),
        preferred_element_type=jnp.float32,
        precision=jax.lax.Precision.DEFAULT,
    )
    s_ref[...] = jax.nn.sigmoid(logits)


def _scores_tc(hf, weight, n_blocks):
    t, h = hf.shape
    return pl.pallas_call(
        _matmul_body,
        grid=(n_blocks,),
        in_specs=[
            pl.BlockSpec((MBLK, h), lambda i: (i, 0)),
            pl.BlockSpec((N_EXPERTS, h), lambda i: (0, 0)),
        ],
        out_specs=pl.BlockSpec((MBLK, N_EXPERTS), lambda i: (i, 0)),
        out_shape=jax.ShapeDtypeStruct((n_blocks * MBLK, N_EXPERTS),
                                       jnp.float32),
        compiler_params=pltpu.CompilerParams(
            dimension_semantics=("arbitrary",),
        ),
    )(hf, weight)


def _take(v, p):
    return jax.lax.gather(v, p[:, None], dimension_numbers=_DNUMS,
                          slice_sizes=(1,), mode=_IB)


def _sc_router(t):
    n_workers = 32
    tpw = t // n_workers
    n_chunks = tpw // CHUNK
    mesh = plsc.VectorSubcoreMesh(core_axis_name="c", subcore_axis_name="s")

    @functools.partial(
        pl.kernel,
        out_type=[
            jax.ShapeDtypeStruct((t * TOP_K,), jnp.int32),
            jax.ShapeDtypeStruct((t * TOP_K,), jnp.float32),
        ],
        mesh=mesh,
        scratch_types=[
            pltpu.VMEM((CHUNK, N_EXPERTS), jnp.float32),
            pltpu.VMEM((64,), jnp.float32),
            pltpu.VMEM((CHUNK * TOP_K + 16,), jnp.int32),
            pltpu.VMEM((CHUNK * TOP_K + 16,), jnp.float32),
        ],
    )
    def route(scores_hbm, bias_hbm, idx_hbm, wgt_hbm, s_v, b_v, i_v, w_v):
        wid = lax.axis_index("s") * 2 + lax.axis_index("c")
        base = wid * tpw

        pltpu.sync_copy(bias_hbm, b_v)
        bias_vregs = [b_v[pl.ds(16 * k, 16)] for k in range(4)]

        iota = lax.iota(jnp.int32, 16)
        iota_f = iota.astype(jnp.float32)
        perms = {d: jnp.bitwise_xor(iota, d) for d in (1, 2, 4, 8)}
        pat08 = (iota & 1) << 3
        half = iota >> 3
        g_of_lane = iota

        def group_top2(v):
            p = _take(v, perms[4])
            a = jnp.maximum(v, p)
            b = jnp.minimum(v, p)
            for d in (2, 1):
                pa = _take(a, perms[d])
                pb = _take(b, perms[d])
                hi = jnp.maximum(a, pa)
                lo = jnp.minimum(a, pa)
                b = jnp.maximum(lo, jnp.maximum(b, pb))
                a = hi
            return a + b

        def token_body(tok, _):
            v = [s_v[tok, pl.ds(16 * k, 16)] for k in range(4)]
            s4c = [v[k] + bias_vregs[k] for k in range(4)]

            gs = [group_top2(s4c[k]) for k in range(4)]
            g8 = _take(gs[0], pat08)
            for k in range(1, 4):
                pk = _take(gs[k], pat08)
                g8 = jnp.where((iota >> 1) == k, pk, g8)

            cnt = jnp.zeros((16,), jnp.float32)
            for k in range(1, N_GROUP):
                pat = (iota + k) & (N_GROUP - 1)
                sg = _take(g8, pat)
                lower = pat < g_of_lane
                beat = (sg > g8) | ((sg == g8) & lower)
                cnt = cnt + jnp.where(beat, 1.0, 0.0)
            selg = jnp.where(cnt < float(TOPK_GROUP), 1.0, 0.0)

            ms = []
            for k in range(4):
                mk = _take(selg, 2 * k + half)
                ms.append(jnp.where(mk > 0.5, s4c[k], 0.0))

            rev = [63.0 - (16.0 * k + iota_f) for k in range(4)]
            res_i = jnp.zeros((16,), jnp.float32)
            res_w = jnp.zeros((16,), jnp.float32)
            for r in range(TOP_K):
                m = jnp.maximum(jnp.maximum(ms[0], ms[1]),
                                jnp.maximum(ms[2], ms[3]))
                for d in (8, 4, 2, 1):
                    m = jnp.maximum(m, _take(m, perms[d]))
                c = jnp.full((16,), NEG, jnp.float32)
                for k in range(4):
                    c = jnp.maximum(c, jnp.where(ms[k] == m, rev[k], NEG))
                for d in (8, 4, 2, 1):
                    c = jnp.maximum(c, _take(c, perms[d]))
                ms = [jnp.where(rev[k] == c, NEG, ms[k]) for k in range(4)]
                res_i = jnp.where(iota == r, 63.0 - c, res_i)
                res_w = jnp.where(iota == r, m, res_w)

            tot = jnp.where(iota < TOP_K, res_w, 0.0)
            for d in (8, 4, 2, 1):
                tot = tot + _take(tot, perms[d])
            res_w = (res_w / (tot + 1e-20)) * SCALE

            i_v[pl.ds(tok * TOP_K, 16)] = res_i.astype(jnp.int32)
            w_v[pl.ds(tok * TOP_K, 16)] = res_w
            return 0

        for ci in range(n_chunks):
            off = base + ci * CHUNK
            pltpu.sync_copy(scores_hbm.at[pl.ds(off, CHUNK)], s_v)
            lax.fori_loop(0, CHUNK, token_body, 0)
            pltpu.sync_copy(i_v.at[pl.ds(0, CHUNK * TOP_K)],
                            idx_hbm.at[pl.ds(off * TOP_K, CHUNK * TOP_K)])
            pltpu.sync_copy(w_v.at[pl.ds(0, CHUNK * TOP_K)],
                            wgt_hbm.at[pl.ds(off * TOP_K, CHUNK * TOP_K)])

    return route


def kernel(hidden_states, weight, e_score_correction_bias):
    b, s, h = hidden_states.shape
    hf = hidden_states.reshape(-1, h).astype(jnp.float32)
    t = hf.shape[0]
    w32 = weight.astype(jnp.float32)
    bias = e_score_correction_bias.astype(jnp.float32)
    bias2 = bias.reshape(1, N_EXPERTS)

    t_sc = SC_TOKENS
    scores_sc = _scores_tc(hf, w32, t_sc // MBLK)
    idx_sc, wgt_sc = _sc_router(t_sc)(scores_sc, bias)
    idx_tc, wgt_tc = _fused_tc(hf, w32, bias2, t_sc // TBLK)

    idx = jax.lax.dynamic_update_slice(
        idx_tc, idx_sc.reshape(t_sc, TOP_K), (0, 0))
    wgt = jax.lax.dynamic_update_slice(
        wgt_tc, wgt_sc.reshape(t_sc, TOP_K), (0, 0))
    return idx, wgt
